# Initial kernel scaffold; baseline (speedup 1.0000x reference)
import functools

import jax
import jax.numpy as jnp
from jax import lax
from jax.experimental import pallas as pl
from jax.experimental.pallas import tpu as pltpu
from jax.experimental.pallas import tpu_sc as plsc

VOCAB = 100000
EMBED_DIM = 64
TOTAL = 4096 * 50

NC = 2
NS = 16
NW = NC * NS
B_PER_W = TOTAL // NW

CHUNK = 128
NCHUNKS = B_PER_W // CHUNK
NBUF = 2

_mesh = plsc.VectorSubcoreMesh(core_axis_name="c", subcore_axis_name="s")


@functools.partial(
    pl.kernel,
    mesh=_mesh,
    out_type=jax.ShapeDtypeStruct((TOTAL, EMBED_DIM), jnp.float32),
    scratch_types=[
        pltpu.VMEM((NCHUNKS, CHUNK), jnp.int32),
        pltpu.VMEM((NBUF, CHUNK, EMBED_DIM), jnp.float32),
        [pltpu.SemaphoreType.DMA for _ in range(NBUF)],
    ],
)
def _gather_kernel(idx_hbm, table_hbm, out_hbm, idx_v, rows_v, gsems):
    wid = lax.axis_index("s") * NC + lax.axis_index("c")
    base = wid * B_PER_W

    pltpu.sync_copy(idx_hbm.at[wid], idx_v)

    for b in range(NBUF):
        pltpu.async_copy(table_hbm.at[idx_v.at[b]], rows_v.at[b], gsems[b])

    @pl.loop(0, NCHUNKS, step=NBUF)
    def _(g):
        for b in range(NBUF):
            ci = g + b
            pltpu.make_async_copy(
                table_hbm.at[idx_v.at[ci]], rows_v.at[b], gsems[b]
            ).wait()
            pltpu.sync_copy(
                rows_v.at[b], out_hbm.at[pl.ds(base + ci * CHUNK, CHUNK)]
            )
            nxt = ci + NBUF

            @pl.when(nxt < NCHUNKS)
            def _():
                pltpu.async_copy(
                    table_hbm.at[idx_v.at[nxt]], rows_v.at[b], gsems[b]
                )


def kernel(x, weights):
    idx = x.reshape(NW, NCHUNKS, CHUNK).astype(jnp.int32)
    out = _gather_kernel(idx, weights)
    return out.reshape(x.shape + (EMBED_DIM,))


# speedup vs baseline: 4.5424x; 4.5424x over previous
import functools

import jax
import jax.numpy as jnp
from jax import lax
from jax.experimental import pallas as pl
from jax.experimental.pallas import tpu as pltpu
from jax.experimental.pallas import tpu_sc as plsc

VOCAB = 100000
EMBED_DIM = 64
TOTAL = 4096 * 50

NC = 2
NS = 16
NW = NC * NS
B_PER_W = TOTAL // NW

CHUNK = 128
NCHUNKS = B_PER_W // CHUNK
NBUF = 2

_mesh = plsc.VectorSubcoreMesh(core_axis_name="c", subcore_axis_name="s")


@functools.partial(
    pl.kernel,
    mesh=_mesh,
    out_type=jax.ShapeDtypeStruct((TOTAL, EMBED_DIM), jnp.float32),
    scratch_types=[
        pltpu.VMEM((NCHUNKS, CHUNK), jnp.int32),
        pltpu.VMEM((NBUF, CHUNK, EMBED_DIM), jnp.float32),
        [pltpu.SemaphoreType.DMA for _ in range(NBUF)],
    ],
    compiler_params=pltpu.CompilerParams(use_tc_tiling_on_sc=False),
)
def _gather_kernel(idx_hbm, table_hbm, out_hbm, idx_v, rows_v, gsems):
    wid = lax.axis_index("s") * NC + lax.axis_index("c")
    base = wid * B_PER_W

    pltpu.sync_copy(idx_hbm.at[wid], idx_v)

    for b in range(NBUF):
        pltpu.async_copy(table_hbm.at[idx_v.at[b]], rows_v.at[b], gsems[b])

    @pl.loop(0, NCHUNKS, step=NBUF)
    def _(g):
        for b in range(NBUF):
            ci = g + b
            pltpu.make_async_copy(
                table_hbm.at[idx_v.at[ci]], rows_v.at[b], gsems[b]
            ).wait()
            pltpu.sync_copy(
                rows_v.at[b], out_hbm.at[pl.ds(base + ci * CHUNK, CHUNK)]
            )
            nxt = ci + NBUF

            @pl.when(nxt < NCHUNKS)
            def _():
                pltpu.async_copy(
                    table_hbm.at[idx_v.at[nxt]], rows_v.at[b], gsems[b]
                )


def kernel(x, weights):
    idx = x.reshape(NW, NCHUNKS, CHUNK).astype(jnp.int32)
    out = _gather_kernel(idx, weights)
    return out.reshape(x.shape + (EMBED_DIM,))


# speedup vs baseline: 4.6699x; 1.0281x over previous
import functools

import jax
import jax.numpy as jnp
from jax import lax
from jax.experimental import pallas as pl
from jax.experimental.pallas import tpu as pltpu
from jax.experimental.pallas import tpu_sc as plsc

VOCAB = 100000
EMBED_DIM = 64
TOTAL = 4096 * 50

NC = 2
NS = 16
NW = NC * NS
B_PER_W = TOTAL // NW

CHUNK = 640
NCHUNKS = B_PER_W // CHUNK
NBUF = 2

_mesh = plsc.VectorSubcoreMesh(core_axis_name="c", subcore_axis_name="s")


@functools.partial(
    pl.kernel,
    mesh=_mesh,
    out_type=jax.ShapeDtypeStruct((TOTAL, EMBED_DIM), jnp.float32),
    scratch_types=[
        pltpu.VMEM((NCHUNKS, CHUNK), jnp.int32),
        pltpu.VMEM((NBUF, CHUNK, EMBED_DIM), jnp.float32),
        [pltpu.SemaphoreType.DMA for _ in range(NBUF)],
    ],
    compiler_params=pltpu.CompilerParams(use_tc_tiling_on_sc=False),
)
def _gather_kernel(idx_hbm, table_hbm, out_hbm, idx_v, rows_v, gsems):
    wid = lax.axis_index("s") * NC + lax.axis_index("c")
    base = wid * B_PER_W

    pltpu.sync_copy(idx_hbm.at[wid], idx_v)

    for b in range(NBUF):
        pltpu.async_copy(table_hbm.at[idx_v.at[b]], rows_v.at[b], gsems[b])

    @pl.loop(0, NCHUNKS, step=NBUF)
    def _(g):
        for b in range(NBUF):
            ci = g + b
            pltpu.make_async_copy(
                table_hbm.at[idx_v.at[ci]], rows_v.at[b], gsems[b]
            ).wait()
            pltpu.sync_copy(
                rows_v.at[b], out_hbm.at[pl.ds(base + ci * CHUNK, CHUNK)]
            )
            nxt = ci + NBUF

            @pl.when(nxt < NCHUNKS)
            def _():
                pltpu.async_copy(
                    table_hbm.at[idx_v.at[nxt]], rows_v.at[b], gsems[b]
                )


def kernel(x, weights):
    idx = x.reshape(NW, NCHUNKS, CHUNK).astype(jnp.int32)
    out = _gather_kernel(idx, weights)
    return out.reshape(x.shape + (EMBED_DIM,))
